# baseline (speedup 1.0000x reference)
import functools

import jax
import jax.numpy as jnp
from jax import lax
from jax.experimental import pallas as pl
from jax.experimental.pallas import tpu as pltpu
from jax.experimental.pallas import tpu_sc as plsc

_NC = 2
_NS = 16
_NW = _NC * _NS
_CH = 128


@functools.lru_cache(maxsize=None)
def _make_gather(V, D, B):
  b_per_w = B // _NW
  nch = b_per_w // _CH
  mesh = plsc.VectorSubcoreMesh(core_axis_name="c", subcore_axis_name="s")

  nbuf = 5

  @functools.partial(
      pl.kernel,
      mesh=mesh,
      out_type=jax.ShapeDtypeStruct((B, D), jnp.float32),
      scratch_types=[
          pltpu.VMEM((nch, _CH), jnp.int32),
          [pltpu.VMEM((_CH, D), jnp.float32) for _ in range(nbuf)],
          [pltpu.SemaphoreType.DMA for _ in range(nbuf)],
          [pltpu.SemaphoreType.DMA for _ in range(nbuf)],
      ],
      compiler_params=pltpu.CompilerParams(use_tc_tiling_on_sc=True),
  )
  def gather(table_hbm, idx_hbm, out_hbm, idx_v, bufs, gsems, wsems):
    wid = lax.axis_index("s") * _NC + lax.axis_index("c")
    base = wid * b_per_w
    pltpu.sync_copy(idx_hbm.at[wid], idx_v)

    def g(j, b):
      return pltpu.make_async_copy(table_hbm.at[idx_v.at[j]], bufs[b],
                                   gsems[b])

    def wr(j, b):
      return pltpu.make_async_copy(
          bufs[b], out_hbm.at[pl.ds(base + j * _CH, _CH)], wsems[b])

    pf = 3
    for b in range(pf):
      g(b, b).start()

    def body(i, carry):
      for b in range(nbuf):
        j = nbuf * i + b
        g(j, b).wait()
        wr(j, b).start()
        jp = j + pf
        bp = (b + pf) % nbuf

        @pl.when(jnp.logical_and(jp < nch, j >= nbuf - pf))
        def _():
          wr(jp, bp).wait()

        @pl.when(jp < nch)
        def _():
          g(jp, bp).start()
      return carry

    lax.fori_loop(0, nch // nbuf, body, 0)
    for b in range(nbuf):
      wr(0, b).wait()

  return gather


def _linear_tc(x, w):
  def mm(x_ref, w_ref, o_ref):
    o_ref[...] = lax.dot_general(
        x_ref[...], w_ref[...], (((1,), (1,)), ((), ())),
        preferred_element_type=jnp.float32)

  return pl.pallas_call(
      mm,
      out_shape=jax.ShapeDtypeStruct((x.shape[0], w.shape[0]), jnp.float32),
  )(x, w)


def kernel(token_ids, dense_feat, embedding_weight, linear_weight):
  B, S = token_ids.shape
  V, D = embedding_weight.shape
  n = B * S
  idx = token_ids.astype(jnp.int32).T.reshape(-1)
  idx3 = idx.reshape(_NW, n // (_NW * _CH), _CH)
  emb_flat = _make_gather(V, D, n)(embedding_weight, idx3)
  emb_out = emb_flat.reshape(S, B, D).transpose(1, 0, 2)
  lin_out = _linear_tc(dense_feat.astype(jnp.float32),
                       linear_weight.astype(jnp.float32))
  return (emb_out, lin_out)
